# baseline (speedup 1.0000x reference)
import functools
import math

import jax
import jax.numpy as jnp
from jax import lax
from jax.experimental import pallas as pl
from jax.experimental.pallas import tpu as pltpu
from jax.experimental.pallas import tpu_sc as plsc

LANES = 16
NUM_CORES = 2
NUM_SUBCORES = 16
NUM_WORKERS = NUM_CORES * NUM_SUBCORES


def _make_kernel(S, B, D, idx_dtype):
    R = S * B
    r_per_w = R // NUM_WORKERS
    s_per_w = S // NUM_WORKERS
    chunk = 32
    n_chunk = r_per_w // chunk
    s_per_chunk = chunk // B
    kv = D // LANES
    scale = math.sqrt(D)

    mesh = plsc.VectorSubcoreMesh(core_axis_name="c", subcore_axis_name="s")

    @functools.partial(
        pl.kernel,
        mesh=mesh,
        out_type=jax.ShapeDtypeStruct((R, D), jnp.float32),
        scratch_types=[
            pltpu.VMEM((n_chunk, chunk), jnp.int32),
            pltpu.VMEM((s_per_w, D), jnp.float32),
            pltpu.VMEM((chunk, D), jnp.float32),
            pltpu.VMEM((chunk, D), jnp.float32),
            pltpu.SemaphoreType.DMA,
            pltpu.SemaphoreType.DMA,
        ],
    )
    def k(x_hbm, pe_hbm, table_hbm, out_hbm, idx_v, pe_v, buf0, buf1, sem0, sem1):
        wid = lax.axis_index("s") * NUM_CORES + lax.axis_index("c")
        rbase = wid * r_per_w

        pltpu.sync_copy(x_hbm.at[wid], idx_v)
        pltpu.sync_copy(pe_hbm.at[wid], pe_v)

        bufs = (buf0, buf1)
        sems = (sem0, sem1)
        copies = [None] * n_chunk
        copies[0] = pltpu.async_copy(table_hbm.at[idx_v.at[0]], bufs[0], sems[0])

        for c in range(n_chunk):
            buf = bufs[c % 2]
            if c + 1 < n_chunk:
                copies[c + 1] = pltpu.async_copy(
                    table_hbm.at[idx_v.at[c + 1]], bufs[(c + 1) % 2], sems[(c + 1) % 2]
                )
            copies[c].wait()

            s_off = c * s_per_chunk

            def sbody(sl, _, buf=buf, s_off=s_off):
                def kbody(j, _, sl=sl, buf=buf):
                    col = j * LANES
                    pv = pe_v[s_off + sl, pl.ds(col, LANES)]
                    for b in range(B):
                        r = sl * B + b
                        buf[r, pl.ds(col, LANES)] = (
                            buf[r, pl.ds(col, LANES)] * scale + pv
                        )
                    return 0

                lax.fori_loop(0, kv, kbody, 0, unroll=4)
                return 0

            lax.fori_loop(0, s_per_chunk, sbody, 0)

            pltpu.sync_copy(buf, out_hbm.at[pl.ds(rbase + c * chunk, chunk)])

    return k


@jax.jit
def kernel(x, table, pe):
    S, B = x.shape
    V, D = table.shape
    x_w = x.astype(jnp.int32).reshape(NUM_WORKERS, -1, 32)
    pe_w = pe[:S].reshape(NUM_WORKERS, -1, D)
    k = _make_kernel(S, B, D, x.dtype)
    out = k(x_w, pe_w, table)
    return out.reshape(S, B, D)
